# baseline (speedup 1.0000x reference)
import functools

import jax
import jax.numpy as jnp
from jax import lax
from jax.experimental import pallas as pl
from jax.experimental.pallas import tpu as pltpu
from jax.experimental.pallas import tpu_sc as plsc


@functools.cache
def _make_gather(V, D, B):
    info = plsc.get_sparse_core_info()
    NC, NS, L = info.num_cores, info.num_subcores, info.num_lanes
    NW = NC * NS
    R = D // NW
    IC = 4096
    NCH = B // IC
    assert D % NW == 0 and B % IC == 0 and IC % L == 0
    mesh = plsc.VectorSubcoreMesh(core_axis_name="c", subcore_axis_name="s")

    @functools.partial(
        pl.kernel,
        mesh=mesh,
        compiler_params=pltpu.CompilerParams(needs_layout_passes=False),
        out_type=jax.ShapeDtypeStruct((D, B), jnp.float32),
        scratch_types=[
            pltpu.VMEM((V,), jnp.float32),
            pltpu.VMEM((B,), jnp.int32),
            pltpu.VMEM((2, IC), jnp.float32),
            pltpu.SemaphoreType.DMA,
            pltpu.SemaphoreType.DMA,
            pltpu.SemaphoreType.DMA,
        ],
    )
    def k(idx_hbm, tabT_hbm, outT_hbm, row_v, idx_v, out_v, s_idx, s_row, s_out):
        wid = lax.axis_index("s") * NC + lax.axis_index("c")
        idx_cp = pltpu.async_copy(idx_hbm, idx_v, s_idx)
        for r in range(R):
            row = wid * R + r
            row_cp = pltpu.async_copy(tabT_hbm.at[row], row_v, s_row)
            if r == 0:
                idx_cp.wait()
            row_cp.wait()
            for ch in range(NCH):
                step = r * NCH + ch
                buf = step % 2
                if step >= 2:
                    pltpu.make_async_copy(
                        out_v.at[buf], outT_hbm.at[row, pl.ds(0, IC)], s_out
                    ).wait()

                U = 8

                @pl.loop(0, IC // (L * U), unroll=2)
                def _gather(g, ch=ch, buf=buf):
                    ivs = [
                        idx_v[pl.ds(ch * IC + (g * U + j) * L, L)]
                        for j in range(U)
                    ]
                    vals = [plsc.load_gather(row_v, [iv]) for iv in ivs]
                    for j in range(U):
                        out_v[buf, pl.ds((g * U + j) * L, L)] = vals[j]

                pltpu.async_copy(
                    out_v.at[buf], outT_hbm.at[row, pl.ds(ch * IC, IC)], s_out
                )
        for buf in range(2):
            pltpu.make_async_copy(
                out_v.at[buf], outT_hbm.at[0, pl.ds(0, IC)], s_out
            ).wait()

    return k


def kernel(indices, table):
    (B,) = indices.shape
    V, D = table.shape
    outT = _make_gather(V, D, B)(indices.astype(jnp.int32), table.T)
    return outT.T


# speedup vs baseline: 1.0211x; 1.0211x over previous
import functools

import jax
import jax.numpy as jnp
from jax import lax
from jax.experimental import pallas as pl
from jax.experimental.pallas import tpu as pltpu
from jax.experimental.pallas import tpu_sc as plsc


@functools.cache
def _make_gather(V, D, B):
    info = plsc.get_sparse_core_info()
    NC, NS, L = info.num_cores, info.num_subcores, info.num_lanes
    NW = NC * NS
    R = D // NW
    IC = 4096
    NCH = B // IC
    assert D % NW == 0 and B % IC == 0 and IC % L == 0
    mesh = plsc.VectorSubcoreMesh(core_axis_name="c", subcore_axis_name="s")

    @functools.partial(
        pl.kernel,
        mesh=mesh,
        compiler_params=pltpu.CompilerParams(needs_layout_passes=False),
        out_type=jax.ShapeDtypeStruct((D, B), jnp.float32),
        scratch_types=[
            pltpu.VMEM((V,), jnp.float32),
            pltpu.VMEM((B,), jnp.int32),
            pltpu.VMEM((2, IC), jnp.float32),
            pltpu.SemaphoreType.DMA,
            pltpu.SemaphoreType.DMA,
            pltpu.SemaphoreType.DMA,
        ],
    )
    def k(idx_hbm, tabT_hbm, outT_hbm, row_v, idx_v, out_v, s_idx, s_row, s_out):
        wid = lax.axis_index("s") * NC + lax.axis_index("c")
        idx_cp = pltpu.async_copy(idx_hbm, idx_v, s_idx)
        for r in range(R):
            row = wid * R + r
            row_cp = pltpu.async_copy(tabT_hbm.at[row], row_v, s_row)
            if r == 0:
                idx_cp.wait()
            row_cp.wait()
            for ch in range(NCH):
                step = r * NCH + ch
                buf = step % 2
                if step >= 2:
                    pltpu.make_async_copy(
                        out_v.at[buf], outT_hbm.at[row, pl.ds(0, IC)], s_out
                    ).wait()

                U = 8

                @pl.loop(0, IC // (L * U))
                def _gather(g, ch=ch, buf=buf):
                    ivs = [
                        idx_v[pl.ds(ch * IC + (g * U + j) * L, L)]
                        for j in range(U)
                    ]
                    vals = [plsc.load_gather(row_v, [iv]) for iv in ivs]
                    for j in range(U):
                        out_v[buf, pl.ds((g * U + j) * L, L)] = vals[j]

                pltpu.async_copy(
                    out_v.at[buf], outT_hbm.at[row, pl.ds(ch * IC, IC)], s_out
                )
        for buf in range(2):
            pltpu.make_async_copy(
                out_v.at[buf], outT_hbm.at[0, pl.ds(0, IC)], s_out
            ).wait()

    return k


def kernel(indices, table):
    (B,) = indices.shape
    V, D = table.shape
    outT = _make_gather(V, D, B)(indices.astype(jnp.int32), table.T)
    return outT.T


# speedup vs baseline: 1.0221x; 1.0010x over previous
import functools

import jax
import jax.numpy as jnp
from jax import lax
from jax.experimental import pallas as pl
from jax.experimental.pallas import tpu as pltpu
from jax.experimental.pallas import tpu_sc as plsc


@functools.cache
def _make_gather(V, D, B):
    info = plsc.get_sparse_core_info()
    NC, NS, L = info.num_cores, info.num_subcores, info.num_lanes
    NW = NC * NS
    R = D // NW
    IC = 4096
    NCH = B // IC
    assert D % NW == 0 and B % IC == 0 and IC % L == 0
    mesh = plsc.VectorSubcoreMesh(core_axis_name="c", subcore_axis_name="s")

    @functools.partial(
        pl.kernel,
        mesh=mesh,
        compiler_params=pltpu.CompilerParams(
            needs_layout_passes=False, skip_device_barrier=True
        ),
        out_type=jax.ShapeDtypeStruct((D, B), jnp.float32),
        scratch_types=[
            pltpu.VMEM((V,), jnp.float32),
            pltpu.VMEM((B,), jnp.int32),
            pltpu.VMEM((2, IC), jnp.float32),
            pltpu.SemaphoreType.DMA,
            pltpu.SemaphoreType.DMA,
            pltpu.SemaphoreType.DMA,
        ],
    )
    def k(idx_hbm, tabT_hbm, outT_hbm, row_v, idx_v, out_v, s_idx, s_row, s_out):
        wid = lax.axis_index("s") * NC + lax.axis_index("c")
        idx_cp = pltpu.async_copy(idx_hbm, idx_v, s_idx)
        for r in range(R):
            row = wid * R + r
            row_cp = pltpu.async_copy(tabT_hbm.at[row], row_v, s_row)
            if r == 0:
                idx_cp.wait()
            row_cp.wait()
            for ch in range(NCH):
                step = r * NCH + ch
                buf = step % 2
                if step >= 2:
                    pltpu.make_async_copy(
                        out_v.at[buf], outT_hbm.at[row, pl.ds(0, IC)], s_out
                    ).wait()

                U = 8

                @pl.loop(0, IC // (L * U))
                def _gather(g, ch=ch, buf=buf):
                    ivs = [
                        idx_v[pl.ds(ch * IC + (g * U + j) * L, L)]
                        for j in range(U)
                    ]
                    vals = [plsc.load_gather(row_v, [iv]) for iv in ivs]
                    for j in range(U):
                        out_v[buf, pl.ds((g * U + j) * L, L)] = vals[j]

                pltpu.async_copy(
                    out_v.at[buf], outT_hbm.at[row, pl.ds(ch * IC, IC)], s_out
                )
        for buf in range(2):
            pltpu.make_async_copy(
                out_v.at[buf], outT_hbm.at[0, pl.ds(0, IC)], s_out
            ).wait()

    return k


def kernel(indices, table):
    (B,) = indices.shape
    V, D = table.shape
    outT = _make_gather(V, D, B)(indices.astype(jnp.int32), table.T)
    return outT.T
